# baseline (speedup 1.0000x reference)
import jax
import jax.numpy as jnp
from jax import lax
from jax.experimental import pallas as pl

_ROWS = 128
_N = 32768
_BR = 16

_INT_MIN = -(2**31)
_INT_MAX = 2**31 - 1


def _count_le(x16, mid16):
    c = (x16 <= mid16).astype(jnp.int16)
    while c.shape[1] > 256:
        h = c.shape[1] // 2
        c = c[:, :h] + c[:, h:]
    return jnp.sum(c.astype(jnp.int32), axis=1, keepdims=True)


def _bisect16(problems):
    states = []
    for x16, _ in problems:
        h = x16.shape[0]
        states.append((jnp.full((h, 1), -(2**15), jnp.int16),
                       jnp.full((h, 1), 2**15 - 1, jnp.int16),
                       jnp.zeros((h, 1), jnp.int32)))
    for _ in range(16):
        nxt = []
        for (x16, kp1), (lo, hi, base) in zip(problems, states):
            s32 = lo.astype(jnp.int32) + hi.astype(jnp.int32)
            mid = lax.shift_right_arithmetic(s32, 1).astype(jnp.int16)
            cnt = _count_le(x16, mid)
            ge = cnt >= kp1
            base = jnp.where(ge, base, cnt)
            m16 = ge.astype(jnp.int32).astype(jnp.int16) > jnp.int16(0)
            one = jnp.ones((), jnp.int16)
            nxt.append((jnp.where(m16, lo, mid + one),
                        jnp.where(m16, mid, hi), base))
        states = nxt
    return [(lo, base) for lo, _, base in states]


def _body(klen_ref, p_ref, g_ref, out_ref):
    conf = jnp.log(p_ref[...]) + g_ref[...]
    b = lax.bitcast_convert_type(conf, jnp.int32)
    key = b ^ (lax.shift_right_arithmetic(b, 31) & jnp.int32(0x7FFFFFFF))

    kp1 = klen_ref[...] + jnp.int32(1)

    top16 = lax.shift_right_arithmetic(key, 16).astype(jnp.int16)
    low16 = ((key & jnp.int32(0xFFFF)) - jnp.int32(32768)).astype(jnp.int16)

    hb = _BR // 2
    (tA, bA), (tB, bB) = _bisect16(
        [(top16[:hb], kp1[:hb]), (top16[hb:], kp1[hb:])])
    t_star = jnp.concatenate([tA, tB], axis=0)
    base = jnp.concatenate([bA, bB], axis=0)

    low_g = jnp.where(top16 == t_star, low16, jnp.full((), 2**15 - 1, jnp.int16))
    (lA, _), (lB, _) = _bisect16(
        [(low_g[:hb], kp1[:hb] - base[:hb]), (low_g[hb:], kp1[hb:] - base[hb:])])
    l_star = jnp.concatenate([lA, lB], axis=0)

    cutoff = (lax.shift_left(t_star.astype(jnp.int32), 16)
              | (l_star.astype(jnp.int32) + jnp.int32(32768)))
    out_ref[...] = key < cutoff


def kernel(mask_len, probs):
    gumbel = jax.random.gumbel(jax.random.key(42), probs.shape, probs.dtype)
    return pl.pallas_call(
        _body,
        grid=(_ROWS // _BR,),
        in_specs=[
            pl.BlockSpec((_BR, 1), lambda i: (i, 0)),
            pl.BlockSpec((_BR, _N), lambda i: (i, 0)),
            pl.BlockSpec((_BR, _N), lambda i: (i, 0)),
        ],
        out_specs=pl.BlockSpec((_BR, _N), lambda i: (i, 0)),
        out_shape=jax.ShapeDtypeStruct((_ROWS, _N), jnp.bool_),
    )(mask_len, probs, gumbel)


# speedup vs baseline: 1.0039x; 1.0039x over previous
import jax
import jax.numpy as jnp
from jax import lax
from jax.experimental import pallas as pl

_ROWS = 128
_N = 32768
_BR = 16

_INT_MIN = -(2**31)
_INT_MAX = 2**31 - 1


def _count_le(le):
    c = le.astype(jnp.int16)
    while c.shape[1] > 256:
        h = c.shape[1] // 2
        c = c[:, :h] + c[:, h:]
    return jnp.sum(c.astype(jnp.int32), axis=1, keepdims=True)


def _bisect16(x16, kp1, gate=None):
    h = x16.shape[0]
    lo = jnp.full((h, 1), -(2**15), jnp.int16)
    hi = jnp.full((h, 1), 2**15 - 1, jnp.int16)
    base = jnp.zeros((h, 1), jnp.int32)
    for _ in range(16):
        s32 = lo.astype(jnp.int32) + hi.astype(jnp.int32)
        mid = lax.shift_right_arithmetic(s32, 1).astype(jnp.int16)
        le = x16 <= mid
        if gate is not None:
            le = le & gate
        cnt = _count_le(le)
        ge = cnt >= kp1
        base = jnp.where(ge, base, cnt)
        m16 = ge.astype(jnp.int32).astype(jnp.int16) > jnp.int16(0)
        one = jnp.ones((), jnp.int16)
        lo, hi = jnp.where(m16, lo, mid + one), jnp.where(m16, mid, hi)
    return lo, base


def _body(klen_ref, p_ref, g_ref, out_ref):
    conf = jnp.log(p_ref[...]) + g_ref[...]
    b = lax.bitcast_convert_type(conf, jnp.int32)
    key = b ^ (lax.shift_right_arithmetic(b, 31) & jnp.int32(0x7FFFFFFF))

    kp1 = klen_ref[...] + jnp.int32(1)

    top16 = lax.shift_right_arithmetic(key, 16).astype(jnp.int16)
    low16 = ((key & jnp.int32(0xFFFF)) - jnp.int32(32768)).astype(jnp.int16)

    t_star, base = _bisect16(top16, kp1)

    l_star, _ = _bisect16(low16, kp1 - base, gate=top16 == t_star)

    cutoff = (lax.shift_left(t_star.astype(jnp.int32), 16)
              | (l_star.astype(jnp.int32) + jnp.int32(32768)))
    out_ref[...] = key < cutoff


def kernel(mask_len, probs):
    gumbel = jax.random.gumbel(jax.random.key(42), probs.shape, probs.dtype)
    return pl.pallas_call(
        _body,
        grid=(_ROWS // _BR,),
        in_specs=[
            pl.BlockSpec((_BR, 1), lambda i: (i, 0)),
            pl.BlockSpec((_BR, _N), lambda i: (i, 0)),
            pl.BlockSpec((_BR, _N), lambda i: (i, 0)),
        ],
        out_specs=pl.BlockSpec((_BR, _N), lambda i: (i, 0)),
        out_shape=jax.ShapeDtypeStruct((_ROWS, _N), jnp.bool_),
    )(mask_len, probs, gumbel)


# speedup vs baseline: 1.1203x; 1.1160x over previous
import jax
import jax.numpy as jnp
from jax import lax
from jax.experimental import pallas as pl

_ROWS = 128
_N = 32768
_BR = 32

_INT_MIN = -(2**31)
_INT_MAX = 2**31 - 1


def _count_le(le):
    c = le.astype(jnp.int16)
    while c.shape[1] > 256:
        h = c.shape[1] // 2
        c = c[:, :h] + c[:, h:]
    return jnp.sum(c.astype(jnp.int32), axis=1, keepdims=True)


def _bisect16(x16, kp1, gate=None):
    h = x16.shape[0]
    lo = jnp.full((h, 1), -(2**15), jnp.int16)
    hi = jnp.full((h, 1), 2**15 - 1, jnp.int16)
    base = jnp.zeros((h, 1), jnp.int32)
    for _ in range(16):
        s32 = lo.astype(jnp.int32) + hi.astype(jnp.int32)
        mid = lax.shift_right_arithmetic(s32, 1).astype(jnp.int16)
        le = x16 <= mid
        if gate is not None:
            le = le & gate
        cnt = _count_le(le)
        ge = cnt >= kp1
        base = jnp.where(ge, base, cnt)
        m16 = ge.astype(jnp.int32).astype(jnp.int16) > jnp.int16(0)
        one = jnp.ones((), jnp.int16)
        lo, hi = jnp.where(m16, lo, mid + one), jnp.where(m16, mid, hi)
    return lo, base


def _body(klen_ref, p_ref, g_ref, out_ref):
    conf = jnp.log(p_ref[...]) + g_ref[...]
    b = lax.bitcast_convert_type(conf, jnp.int32)
    key = b ^ (lax.shift_right_arithmetic(b, 31) & jnp.int32(0x7FFFFFFF))

    kp1 = klen_ref[...] + jnp.int32(1)

    top16 = lax.shift_right_arithmetic(key, 16).astype(jnp.int16)
    low16 = ((key & jnp.int32(0xFFFF)) - jnp.int32(32768)).astype(jnp.int16)

    t_star, base = _bisect16(top16, kp1)

    l_star, _ = _bisect16(low16, kp1 - base, gate=top16 == t_star)

    cutoff = (lax.shift_left(t_star.astype(jnp.int32), 16)
              | (l_star.astype(jnp.int32) + jnp.int32(32768)))
    out_ref[...] = key < cutoff


def kernel(mask_len, probs):
    gumbel = jax.random.gumbel(jax.random.key(42), probs.shape, probs.dtype)
    return pl.pallas_call(
        _body,
        grid=(_ROWS // _BR,),
        in_specs=[
            pl.BlockSpec((_BR, 1), lambda i: (i, 0)),
            pl.BlockSpec((_BR, _N), lambda i: (i, 0)),
            pl.BlockSpec((_BR, _N), lambda i: (i, 0)),
        ],
        out_specs=pl.BlockSpec((_BR, _N), lambda i: (i, 0)),
        out_shape=jax.ShapeDtypeStruct((_ROWS, _N), jnp.bool_),
    )(mask_len, probs, gumbel)


# speedup vs baseline: 1.1219x; 1.0014x over previous
import jax
import jax.numpy as jnp
from jax import lax
from jax.experimental import pallas as pl

_ROWS = 128
_N = 32768
_BR = 32


def _count_le(le):
    c = le.astype(jnp.int16)
    while c.shape[1] > 256:
        h = c.shape[1] // 2
        c = c[:, :h] + c[:, h:]
    return jnp.sum(c.astype(jnp.int32), axis=1, keepdims=True)


def _bisect16(x16, kp1, gate=None):
    h = x16.shape[0]
    lo = jnp.full((h, 1), -(2**15), jnp.int16)
    hi = jnp.full((h, 1), 2**15 - 1, jnp.int16)
    base = jnp.zeros((h, 1), jnp.int32)
    for _ in range(16):
        s32 = lo.astype(jnp.int32) + hi.astype(jnp.int32)
        mid = lax.shift_right_arithmetic(s32, 1).astype(jnp.int16)
        le = x16 <= mid
        if gate is not None:
            le = le & gate
        cnt = _count_le(le)
        ge = cnt >= kp1
        base = jnp.where(ge, base, cnt)
        m16 = ge.astype(jnp.int32).astype(jnp.int16) > jnp.int16(0)
        one = jnp.ones((), jnp.int16)
        lo, hi = jnp.where(m16, lo, mid + one), jnp.where(m16, mid, hi)
    return lo, base


def _body(klen_ref, p_ref, g_ref, out_ref):
    conf = jnp.log(p_ref[...]) + g_ref[...]
    b = lax.bitcast_convert_type(conf, jnp.int32)
    key = b ^ (lax.shift_right_arithmetic(b, 31) & jnp.int32(0x7FFFFFFF))

    kp1 = klen_ref[...] + jnp.int32(1)

    top16 = lax.shift_right_arithmetic(key, 16).astype(jnp.int16)
    low16 = ((key & jnp.int32(0xFFFF)) - jnp.int32(32768)).astype(jnp.int16)

    t_star, base = _bisect16(top16, kp1)

    l_star, _ = _bisect16(low16, kp1 - base, gate=top16 == t_star)

    cutoff = (lax.shift_left(t_star.astype(jnp.int32), 16)
              | (l_star.astype(jnp.int32) + jnp.int32(32768)))
    out_ref[...] = key < cutoff


def kernel(mask_len, probs):
    gumbel = jax.random.gumbel(jax.random.key(42), probs.shape, probs.dtype)
    return pl.pallas_call(
        _body,
        grid=(_ROWS // _BR,),
        in_specs=[
            pl.BlockSpec((_BR, 1), lambda i: (i, 0)),
            pl.BlockSpec((_BR, _N), lambda i: (i, 0)),
            pl.BlockSpec((_BR, _N), lambda i: (i, 0)),
        ],
        out_specs=pl.BlockSpec((_BR, _N), lambda i: (i, 0)),
        out_shape=jax.ShapeDtypeStruct((_ROWS, _N), jnp.bool_),
    )(mask_len, probs, gumbel)
